# baseline (speedup 1.0000x reference)
import functools

import jax
import jax.numpy as jnp
from jax import lax
from jax.experimental import pallas as pl
from jax.experimental.pallas import tpu as pltpu
from jax.experimental.pallas import tpu_sc as plsc

F32 = jnp.float32

_N = 10000
_C = 128
_E = 160000
_L = 3
_ED = 16
_NREL = 8

_NC = 2
_NS = 16
_CH = _C // _NC
_EPT = _E // _NS
_BE = 128
_NB = 80
_NPAD = 10240
_ZROW = 128


def _pre_body(first, hl_ref, w_ref, as_w_ref, ad_w_ref, ae_w_ref, we_ref,
              emb_ref, h2_ref, aso_ref, ado_ref, relm_ref):
    if first:
        colsum = jnp.sum(w_ref[...], axis=0, keepdims=True)
        h = hl_ref[...] * colsum
    else:
        h = jnp.dot(hl_ref[...], w_ref[...],
                    preferred_element_type=F32)
    h2_ref[0:_N, :] = h[:, 0:_CH]
    h2_ref[_N:2 * _N, :] = h[:, _CH:_C]
    a_s = jnp.sum(h * as_w_ref[...], axis=1, keepdims=True)
    a_d = jnp.sum(h * ad_w_ref[...], axis=1, keepdims=True)
    aso_ref[0:_N, :] = a_s
    ado_ref[0:_N, :] = a_d
    aso_ref[_N:_NPAD, :] = jnp.zeros((_NPAD - _N, 1), F32)
    ado_ref[_N:_NPAD, :] = jnp.zeros((_NPAD - _N, 1), F32)
    wvec = jnp.sum(we_ref[...] * ae_w_ref[...], axis=1)
    rel = jnp.sum(emb_ref[...] * wvec[None, :], axis=1)
    m = jnp.maximum(jnp.max(a_s) + jnp.max(a_d) + jnp.max(rel), 0.0)
    vec = jnp.concatenate(
        [rel, jnp.zeros((16 - _NREL - 1,), F32), m[None]])
    relm_ref[...] = vec[None, :]


def _pre_call(first, hl, w, as_w, ad_w, ae_w, we, emb):
    return pl.pallas_call(
        functools.partial(_pre_body, first),
        out_shape=[
            jax.ShapeDtypeStruct((2 * _N, _CH), F32),
            jax.ShapeDtypeStruct((_NPAD, 1), F32),
            jax.ShapeDtypeStruct((_NPAD, 1), F32),
            jax.ShapeDtypeStruct((1, 16), F32),
        ],
    )(hl, w, as_w, ad_w, ae_w, we, emb)


@functools.cache
def _edge_kernel_build():
  mesh = plsc.VectorSubcoreMesh(core_axis_name="c", subcore_axis_name="s",
                                num_cores=_NC, num_subcores=_NS)

  @functools.partial(
    pl.kernel,
    out_type=[
        jax.ShapeDtypeStruct((2 * _NPAD, _CH), F32),
        jax.ShapeDtypeStruct((2 * _NPAD,), F32),
    ],
    mesh=mesh,
    compiler_params=pltpu.CompilerParams(needs_layout_passes=False,
                                         use_tc_tiling_on_sc=False),
    scratch_types=[
        pltpu.VMEM((_NB, _BE), jnp.int32),
        pltpu.VMEM((_NB, _BE), jnp.int32),
        pltpu.VMEM((_NB, _BE), jnp.int32),
        pltpu.VMEM((_NB, _BE), F32),
        pltpu.VMEM((_NPAD,), F32),
        pltpu.VMEM((_NPAD,), F32),
        pltpu.VMEM((16,), F32),
        pltpu.VMEM((_BE, _CH), F32),
        pltpu.VMEM((_BE, _CH), F32),
        pltpu.VMEM((_ZROW, _CH), F32),
        pltpu.VMEM((640,), F32),
        pltpu.VMEM_SHARED((_NPAD, _CH), F32),
        pltpu.VMEM_SHARED((_NPAD,), F32),
        pltpu.SemaphoreType.DMA,
        pltpu.SemaphoreType.DMA,
      ],
  )
  def _edge_kernel(srcp, dstp, etp, as_hbm, ad_hbm, relm_hbm, h2_hbm,
                   acc_out, den_out, src_v, dst_v, et_v, ea_v, as_v, ad_v,
                   relm_v, rows_a, rows_b, zrow, zvec, acc_sh, den_sh,
                   sem_a, sem_b):
      cid = lax.axis_index("c")
      sid = lax.axis_index("s")

      zero16 = jnp.zeros((16,), F32)

      def _zrow_body(r, _):
          for c8 in range(_CH // 16):
              zrow[r, pl.ds(c8 * 16, 16)] = zero16
          return 0
      lax.fori_loop(0, _ZROW, _zrow_body, 0)
      for k in range(640 // 16):
          zvec[pl.ds(k * 16, 16)] = zero16
      for b in range(5):
          off = sid * 640 + b * _ZROW
          pltpu.sync_copy(zrow, acc_sh.at[pl.ds(off, _ZROW)])
      pltpu.sync_copy(zvec, den_sh.at[pl.ds(sid * 640, 640)])

      pltpu.sync_copy(srcp.at[sid], src_v)
      pltpu.sync_copy(dstp.at[sid], dst_v)
      pltpu.sync_copy(etp.at[sid], et_v)
      pltpu.sync_copy(as_hbm, as_v)
      pltpu.sync_copy(ad_hbm, ad_v)
      pltpu.sync_copy(relm_hbm, relm_v)

      plsc.subcore_barrier()

      m = relm_v[...][15]
      roff = cid * _N

      def _p1(j):
          for k in range(_BE // 16):
              sl = pl.ds(k * 16, 16)
              sv = src_v[j, sl]
              dv = dst_v[j, sl]
              tv = et_v[j, sl]
              a = (plsc.load_gather(as_v, [sv])
                   + plsc.load_gather(ad_v, [dv])
                   + plsc.load_gather(relm_v, [tv]))
              a = jnp.where(a > 0.0, a, 0.2 * a)
              ea = jnp.exp(a - m)
              pos = j * _BE + k * 16 + lax.iota(jnp.int32, 16)
              ea_v[j, sl] = jnp.where(pos < _EPT, ea, 0.0)
              src_v[j, sl] = sv + roff

      def _proc(j, rows):
          def _scale(g, _):
              ev = ea_v[j, pl.ds(g * 16, 16)]
              for i in range(16):
                  c = ev[i]
                  r = g * 16 + i
                  for c8 in range(_CH // 16):
                      sl = pl.ds(c8 * 16, 16)
                      rows[r, sl] = rows[r, sl] * c
              return 0
          lax.fori_loop(0, _BE // 16, _scale, 0)
          pltpu.sync_copy(rows, acc_sh.at[dst_v.at[j]], add=True)
          pltpu.sync_copy(ea_v.at[j], den_sh.at[dst_v.at[j]], add=True)

      _p1(0)
      pltpu.async_copy(h2_hbm.at[src_v.at[0]], rows_a, sem_a)

      def _p2_body(jj, _):
          j0 = 2 * jj
          j1 = j0 + 1
          _p1(j1)
          pltpu.async_copy(h2_hbm.at[src_v.at[j1]], rows_b, sem_b)
          pltpu.make_async_copy(
              h2_hbm.at[pl.ds(0, _BE)], rows_a, sem_a).wait()
          _proc(j0, rows_a)

          @pl.when(jj + 1 < _NB // 2)
          def _prefetch():
              _p1(j0 + 2)
              pltpu.async_copy(h2_hbm.at[src_v.at[j0 + 2]], rows_a, sem_a)

          pltpu.make_async_copy(
              h2_hbm.at[pl.ds(0, _BE)], rows_b, sem_b).wait()
          _proc(j1, rows_b)
          return 0
      lax.fori_loop(0, _NB // 2, _p2_body, 0)

      plsc.subcore_barrier()

      for b in range(5):
          off = sid * 640 + b * _ZROW
          pltpu.sync_copy(acc_sh.at[pl.ds(off, _ZROW)],
                          acc_out.at[pl.ds(cid * _NPAD + off, _ZROW)])
      pltpu.sync_copy(den_sh.at[pl.ds(sid * 640, 640)],
                      den_out.at[pl.ds(cid * _NPAD + sid * 640, 640)])

  return _edge_kernel


def _post_body(has_res, *refs):
    if has_res:
        (accp_ref, denp_ref, bias_ref, hl_ref, nw_ref, nb_ref, o_ref) = refs
    else:
        (accp_ref, denp_ref, bias_ref, nw_ref, nb_ref, o_ref) = refs
    acc = jnp.concatenate(
        [accp_ref[0, 0:_N, :], accp_ref[1, 0:_N, :]], axis=1)
    den = denp_ref[0, 0:_N, :]
    o = acc / (den + 1e-16) + bias_ref[...]
    if has_res:
        o = o + hl_ref[...]
    o = o - jnp.mean(o)
    o = o / (jnp.sqrt(jnp.mean(o * o)) + 1e-5)
    o = o * nw_ref[...] + nb_ref[...]
    o_ref[...] = jnp.maximum(o, 0.0)


def _post_call(accp, denp, bias, hl, nw, nb):
    has_res = hl is not None
    args = (accp, denp, bias) + ((hl,) if has_res else ()) + (nw, nb)
    return pl.pallas_call(
        functools.partial(_post_body, has_res),
        out_shape=jax.ShapeDtypeStruct((_N, _C), F32),
    )(*args)


def _mlp_body(h_ref, w1_ref, b1_ref, w2_ref, b2_ref, o_ref):
    z = jnp.dot(h_ref[...], w1_ref[...], preferred_element_type=F32)
    z = jnp.maximum(z + b1_ref[...], 0.0)
    z = jnp.dot(z, w2_ref[...], preferred_element_type=F32) + b2_ref[...]
    o_ref[...] = 1.0 / (1.0 + jnp.exp(-z))


def _mlp_call(h, w1, b1, w2, b2):
    return pl.pallas_call(
        _mlp_body,
        out_shape=jax.ShapeDtypeStruct((_N, 1), F32),
    )(h, w1, b1, w2, b2)


def _pad_edges(a):
    return jnp.pad(a.reshape(_NS, _EPT),
                   ((0, 0), (0, _NB * _BE - _EPT))).reshape(_NS, _NB, _BE)


def kernel(x, edge_index, edge_type, lin_W, lin_edge_W, att_src, att_dst,
           att_edge, conv_bias, edge_emb, norm_weight, norm_bias,
           mlp_W1, mlp_b1, mlp_W2, mlp_b2):
    srcp = _pad_edges(edge_index[0])
    dstp = _pad_edges(edge_index[1])
    etp = _pad_edges(edge_type)
    nw = norm_weight[None, :]
    nb = norm_bias[None, :]

    h = x
    for i in range(_L):
        hl = h
        h2, aso, ado, relm = _pre_call(
            i == 0, hl, lin_W[i], att_src[i][None, :], att_dst[i][None, :],
            att_edge[i][None, :], lin_edge_W[i], edge_emb[i])
        acc2, den2 = _edge_kernel_build()(
            srcp, dstp, etp, aso.reshape(_NPAD), ado.reshape(_NPAD),
            relm.reshape(16), h2)
        h = _post_call(acc2.reshape(2, _NPAD, _CH),
                       den2.reshape(2, _NPAD, 1),
                       conv_bias[i][None, :], hl if i > 0 else None, nw, nb)
    return _mlp_call(h, mlp_W1, mlp_b1[None, :], mlp_W2, mlp_b2[None, :])


# speedup vs baseline: 1.5634x; 1.5634x over previous
import functools

import jax
import jax.numpy as jnp
from jax import lax
from jax.experimental import pallas as pl
from jax.experimental.pallas import tpu as pltpu
from jax.experimental.pallas import tpu_sc as plsc

F32 = jnp.float32

_N = 10000
_C = 128
_E = 160000
_L = 3
_ED = 16
_NREL = 8

_NC = 2
_NS = 16
_CH = _C // _NC
_EPT = _E // _NS
_BE = 128
_NB = 80
_NBC = 8
_NCH = _NB // _NBC
_NPAD = 10240
_ZROW = 128


def _pre_body(first, hl_ref, w_ref, as_w_ref, ad_w_ref, ae_w_ref, we_ref,
              emb_ref, h2_ref, aso_ref, ado_ref, relm_ref):
    if first:
        colsum = jnp.sum(w_ref[...], axis=0, keepdims=True)
        h = hl_ref[...] * colsum
    else:
        h = jnp.dot(hl_ref[...], w_ref[...],
                    preferred_element_type=F32)
    h2_ref[0:_N, :] = h[:, 0:_CH]
    h2_ref[_N:_NPAD, :] = jnp.zeros((_NPAD - _N, _CH), F32)
    h2_ref[_NPAD:_NPAD + _N, :] = h[:, _CH:_C]
    h2_ref[_NPAD + _N:2 * _NPAD, :] = jnp.zeros((_NPAD - _N, _CH), F32)
    a_s = jnp.sum(h * as_w_ref[...], axis=1, keepdims=True)
    a_d = jnp.sum(h * ad_w_ref[...], axis=1, keepdims=True)
    aso_ref[0:_N, :] = a_s
    ado_ref[0:_N, :] = a_d
    aso_ref[_N:_NPAD, :] = jnp.zeros((_NPAD - _N, 1), F32)
    ado_ref[_N:_NPAD, :] = jnp.zeros((_NPAD - _N, 1), F32)
    wvec = jnp.sum(we_ref[...] * ae_w_ref[...], axis=1)
    rel = jnp.sum(emb_ref[...] * wvec[None, :], axis=1)
    m = jnp.maximum(jnp.max(a_s) + jnp.max(a_d) + jnp.max(rel), 0.0)
    vec = jnp.concatenate(
        [rel, jnp.zeros((16 - _NREL - 1,), F32), m[None]])
    relm_ref[...] = vec[None, :]


def _pre_call(first, hl, w, as_w, ad_w, ae_w, we, emb):
    return pl.pallas_call(
        functools.partial(_pre_body, first),
        out_shape=[
            jax.ShapeDtypeStruct((2 * _NPAD, _CH), F32),
            jax.ShapeDtypeStruct((_NPAD, 1), F32),
            jax.ShapeDtypeStruct((_NPAD, 1), F32),
            jax.ShapeDtypeStruct((1, 16), F32),
        ],
    )(hl, w, as_w, ad_w, ae_w, we, emb)


@functools.cache
def _edge_kernel_build():
  mesh = plsc.VectorSubcoreMesh(core_axis_name="c", subcore_axis_name="s",
                                num_cores=_NC, num_subcores=_NS)

  @functools.partial(
    pl.kernel,
    out_type=[
        jax.ShapeDtypeStruct((2 * _NPAD, _CH), F32),
        jax.ShapeDtypeStruct((2 * _NPAD,), F32),
    ],
    mesh=mesh,
    compiler_params=pltpu.CompilerParams(needs_layout_passes=False,
                                         use_tc_tiling_on_sc=False),
    scratch_types=[
        pltpu.VMEM((2, _NBC, _BE), jnp.int32),
        pltpu.VMEM((2, _NBC, _BE), jnp.int32),
        pltpu.VMEM((2, _NBC, _BE), jnp.int32),
        pltpu.VMEM((2, _NBC, _BE), F32),
        pltpu.VMEM((16,), F32),
        pltpu.VMEM((_NPAD,), F32),
        pltpu.VMEM((_NPAD,), F32),
        pltpu.VMEM((_BE, _CH), F32),
        pltpu.VMEM((_BE, _CH), F32),
        pltpu.VMEM((640,), F32),
        pltpu.VMEM_SHARED((_NPAD, _CH), F32),
        pltpu.VMEM_SHARED((_NPAD, _CH), F32),
        pltpu.VMEM_SHARED((_NPAD,), F32),
        pltpu.SemaphoreType.DMA,
        pltpu.SemaphoreType.DMA,
        pltpu.SemaphoreType.DMA,
        pltpu.SemaphoreType.DMA,
        pltpu.SemaphoreType.DMA,
      ],
  )
  def _edge_kernel(srcp, dstp, etp, as_hbm, ad_hbm, relm_hbm, h2_hbm,
                   acc_out, den_out, src_v, dst_v, et_v, ea_v, relm_v,
                   as_v, ad_v, rows_a, rows_b, zvec, h2_sh, acc_sh,
                   den_sh, sem_a, sem_b, sem_s0, sem_s1, sem_s2):
      cid = lax.axis_index("c")
      sid = lax.axis_index("s")

      zero16 = jnp.zeros((16,), F32)

      def _zrow_body(r, _):
          for c8 in range(_CH // 16):
              rows_a[r, pl.ds(c8 * 16, 16)] = zero16
          return 0
      lax.fori_loop(0, _ZROW, _zrow_body, 0)
      for k in range(640 // 16):
          zvec[pl.ds(k * 16, 16)] = zero16
      for b in range(5):
          off = sid * 640 + b * _ZROW
          pltpu.sync_copy(rows_a, acc_sh.at[pl.ds(off, _ZROW)])
      pltpu.sync_copy(zvec, den_sh.at[pl.ds(sid * 640, 640)])

      hoff = cid * _NPAD + sid * 640
      pltpu.sync_copy(h2_hbm.at[pl.ds(hoff, 640)],
                      h2_sh.at[pl.ds(sid * 640, 640)])
      pltpu.sync_copy(as_hbm, as_v)
      pltpu.sync_copy(ad_hbm, ad_v)
      pltpu.sync_copy(relm_hbm, relm_v)

      pltpu.sync_copy(srcp.at[sid, pl.ds(0, _NBC)], src_v.at[0])
      pltpu.sync_copy(dstp.at[sid, pl.ds(0, _NBC)], dst_v.at[0])
      pltpu.sync_copy(etp.at[sid, pl.ds(0, _NBC)], et_v.at[0])

      plsc.subcore_barrier()

      m = relm_v[...][15]

      def _p1(p, c, j):
          for k in range(_BE // 16):
              sl = pl.ds(k * 16, 16)
              sv = src_v[p, j, sl]
              dv = dst_v[p, j, sl]
              tv = et_v[p, j, sl]
              a = (plsc.load_gather(as_v, [sv])
                   + plsc.load_gather(ad_v, [dv])
                   + plsc.load_gather(relm_v, [tv]))
              a = jnp.where(a > 0.0, a, 0.2 * a)
              ea = jnp.exp(a - m)
              pos = (c * _NBC + j) * _BE + k * 16 + lax.iota(jnp.int32, 16)
              ea_v[p, j, sl] = jnp.where(pos < _EPT, ea, 0.0)

      def _proc(p, j, rows):
          def _scale(g, _):
              ev = ea_v[p, j, pl.ds(g * 16, 16)]
              for i in range(16):
                  cc = ev[i]
                  r = g * 16 + i
                  for c8 in range(_CH // 16):
                      sl = pl.ds(c8 * 16, 16)
                      rows[r, sl] = rows[r, sl] * cc
              return 0
          lax.fori_loop(0, _BE // 16, _scale, 0)
          pltpu.sync_copy(rows, acc_sh.at[dst_v.at[p, j]], add=True)
          pltpu.sync_copy(ea_v.at[p, j], den_sh.at[dst_v.at[p, j]], add=True)

      for c in range(_NCH):
          p = c % 2
          if c + 1 < _NCH:
              q = 1 - p
              nsl = pl.ds((c + 1) * _NBC, _NBC)
              pltpu.async_copy(srcp.at[sid, nsl], src_v.at[q], sem_s0)
              pltpu.async_copy(dstp.at[sid, nsl], dst_v.at[q], sem_s1)
              pltpu.async_copy(etp.at[sid, nsl], et_v.at[q], sem_s2)

          def _p1_body(j, _):
              _p1(p, c, j)
              return 0
          lax.fori_loop(0, _NBC, _p1_body, 0)

          pltpu.async_copy(h2_sh.at[src_v.at[p, 0]], rows_a, sem_a)

          def _pipe_body(jj, _):
              j0 = 2 * jj
              j1 = j0 + 1
              pltpu.async_copy(h2_sh.at[src_v.at[p, j1]], rows_b, sem_b)
              pltpu.make_async_copy(
                  h2_sh.at[pl.ds(0, _BE)], rows_a, sem_a).wait()
              _proc(p, j0, rows_a)

              @pl.when(jj + 1 < _NBC // 2)
              def _prefetch():
                  pltpu.async_copy(
                      h2_sh.at[src_v.at[p, j0 + 2]], rows_a, sem_a)

              pltpu.make_async_copy(
                  h2_sh.at[pl.ds(0, _BE)], rows_b, sem_b).wait()
              _proc(p, j1, rows_b)
              return 0
          lax.fori_loop(0, _NBC // 2, _pipe_body, 0)

          if c + 1 < _NCH:
              nsl = pl.ds((c + 1) * _NBC, _NBC)
              q = 1 - p
              pltpu.make_async_copy(
                  srcp.at[sid, nsl], src_v.at[q], sem_s0).wait()
              pltpu.make_async_copy(
                  dstp.at[sid, nsl], dst_v.at[q], sem_s1).wait()
              pltpu.make_async_copy(
                  etp.at[sid, nsl], et_v.at[q], sem_s2).wait()

      plsc.subcore_barrier()

      for b in range(5):
          off = sid * 640 + b * _ZROW
          pltpu.sync_copy(acc_sh.at[pl.ds(off, _ZROW)],
                          acc_out.at[pl.ds(cid * _NPAD + off, _ZROW)])
      pltpu.sync_copy(den_sh.at[pl.ds(sid * 640, 640)],
                      den_out.at[pl.ds(cid * _NPAD + sid * 640, 640)])

  return _edge_kernel


def _post_body(has_res, *refs):
    if has_res:
        (accp_ref, denp_ref, bias_ref, hl_ref, nw_ref, nb_ref, o_ref) = refs
    else:
        (accp_ref, denp_ref, bias_ref, nw_ref, nb_ref, o_ref) = refs
    acc = jnp.concatenate(
        [accp_ref[0, 0:_N, :], accp_ref[1, 0:_N, :]], axis=1)
    den = denp_ref[0, 0:_N, :]
    o = acc / (den + 1e-16) + bias_ref[...]
    if has_res:
        o = o + hl_ref[...]
    o = o - jnp.mean(o)
    o = o / (jnp.sqrt(jnp.mean(o * o)) + 1e-5)
    o = o * nw_ref[...] + nb_ref[...]
    o_ref[...] = jnp.maximum(o, 0.0)


def _post_call(accp, denp, bias, hl, nw, nb):
    has_res = hl is not None
    args = (accp, denp, bias) + ((hl,) if has_res else ()) + (nw, nb)
    return pl.pallas_call(
        functools.partial(_post_body, has_res),
        out_shape=jax.ShapeDtypeStruct((_N, _C), F32),
    )(*args)


def _mlp_body(h_ref, w1_ref, b1_ref, w2_ref, b2_ref, o_ref):
    z = jnp.dot(h_ref[...], w1_ref[...], preferred_element_type=F32)
    z = jnp.maximum(z + b1_ref[...], 0.0)
    z = jnp.dot(z, w2_ref[...], preferred_element_type=F32) + b2_ref[...]
    o_ref[...] = 1.0 / (1.0 + jnp.exp(-z))


def _mlp_call(h, w1, b1, w2, b2):
    return pl.pallas_call(
        _mlp_body,
        out_shape=jax.ShapeDtypeStruct((_N, 1), F32),
    )(h, w1, b1, w2, b2)


def _pad_edges(a):
    return jnp.pad(a.reshape(_NS, _EPT),
                   ((0, 0), (0, _NB * _BE - _EPT))).reshape(_NS, _NB, _BE)


def kernel(x, edge_index, edge_type, lin_W, lin_edge_W, att_src, att_dst,
           att_edge, conv_bias, edge_emb, norm_weight, norm_bias,
           mlp_W1, mlp_b1, mlp_W2, mlp_b2):
    srcp = _pad_edges(edge_index[0])
    dstp = _pad_edges(edge_index[1])
    etp = _pad_edges(edge_type)
    nw = norm_weight[None, :]
    nb = norm_bias[None, :]

    h = x
    for i in range(_L):
        hl = h
        h2, aso, ado, relm = _pre_call(
            i == 0, hl, lin_W[i], att_src[i][None, :], att_dst[i][None, :],
            att_edge[i][None, :], lin_edge_W[i], edge_emb[i])
        acc2, den2 = _edge_kernel_build()(
            srcp, dstp, etp, aso.reshape(_NPAD), ado.reshape(_NPAD),
            relm.reshape(16), h2)
        h = _post_call(acc2.reshape(2, _NPAD, _CH),
                       den2.reshape(2, _NPAD, 1),
                       conv_bias[i][None, :], hl if i > 0 else None, nw, nb)
    return _mlp_call(h, mlp_W1, mlp_b1[None, :], mlp_W2, mlp_b2[None, :])


# speedup vs baseline: 1.5850x; 1.0138x over previous
import functools

import jax
import jax.numpy as jnp
from jax import lax
from jax.experimental import pallas as pl
from jax.experimental.pallas import tpu as pltpu
from jax.experimental.pallas import tpu_sc as plsc

F32 = jnp.float32

_N = 10000
_C = 128
_E = 160000
_L = 3
_ED = 16
_NREL = 8

_NC = 2
_NS = 16
_CH = _C // _NC
_EPT = _E // _NS
_BE = 128
_NB = 80
_NBC = 8
_NCH = _NB // _NBC
_NPAD = 10240
_ZROW = 128


def _pre_body(first, hl_ref, w_ref, as_w_ref, ad_w_ref, ae_w_ref, we_ref,
              emb_ref, h2_ref, aso_ref, ado_ref, relm_ref):
    if first:
        colsum = jnp.sum(w_ref[...], axis=0, keepdims=True)
        h = hl_ref[...] * colsum
    else:
        h = jnp.dot(hl_ref[...], w_ref[...],
                    preferred_element_type=F32)
    h2_ref[0:_N, :] = h[:, 0:_CH]
    h2_ref[_N:_NPAD, :] = jnp.zeros((_NPAD - _N, _CH), F32)
    h2_ref[_NPAD:_NPAD + _N, :] = h[:, _CH:_C]
    h2_ref[_NPAD + _N:2 * _NPAD, :] = jnp.zeros((_NPAD - _N, _CH), F32)
    a_s = jnp.sum(h * as_w_ref[...], axis=1, keepdims=True)
    a_d = jnp.sum(h * ad_w_ref[...], axis=1, keepdims=True)
    aso_ref[0:_N, :] = a_s
    ado_ref[0:_N, :] = a_d
    aso_ref[_N:_NPAD, :] = jnp.zeros((_NPAD - _N, 1), F32)
    ado_ref[_N:_NPAD, :] = jnp.zeros((_NPAD - _N, 1), F32)
    wvec = jnp.sum(we_ref[...] * ae_w_ref[...], axis=1)
    rel = jnp.sum(emb_ref[...] * wvec[None, :], axis=1)
    m = jnp.maximum(jnp.max(a_s) + jnp.max(a_d) + jnp.max(rel), 0.0)
    vec = jnp.concatenate(
        [rel, jnp.zeros((16 - _NREL - 1,), F32), m[None]])
    relm_ref[...] = vec[None, :]


def _pre_call(first, hl, w, as_w, ad_w, ae_w, we, emb):
    return pl.pallas_call(
        functools.partial(_pre_body, first),
        out_shape=[
            jax.ShapeDtypeStruct((2 * _NPAD, _CH), F32),
            jax.ShapeDtypeStruct((_NPAD, 1), F32),
            jax.ShapeDtypeStruct((_NPAD, 1), F32),
            jax.ShapeDtypeStruct((1, 16), F32),
        ],
    )(hl, w, as_w, ad_w, ae_w, we, emb)


@functools.cache
def _edge_kernel_build():
  mesh = plsc.VectorSubcoreMesh(core_axis_name="c", subcore_axis_name="s",
                                num_cores=_NC, num_subcores=_NS)

  @functools.partial(
    pl.kernel,
    out_type=[
        jax.ShapeDtypeStruct((2 * _NPAD, _CH), F32),
        jax.ShapeDtypeStruct((2 * _NPAD,), F32),
    ],
    mesh=mesh,
    compiler_params=pltpu.CompilerParams(needs_layout_passes=False,
                                         use_tc_tiling_on_sc=False),
    scratch_types=[
        pltpu.VMEM((2, _NBC, _BE), jnp.int32),
        pltpu.VMEM((2, _NBC, _BE), jnp.int32),
        pltpu.VMEM((2, _NBC, _BE), jnp.int32),
        pltpu.VMEM((2, _NBC, _BE), F32),
        pltpu.VMEM((16,), F32),
        pltpu.VMEM((_NPAD,), F32),
        pltpu.VMEM((_NPAD,), F32),
        pltpu.VMEM((_BE, _CH), F32),
        pltpu.VMEM((_BE, _CH), F32),
        pltpu.VMEM((640,), F32),
        pltpu.VMEM_SHARED((_NPAD, _CH), F32),
        pltpu.VMEM_SHARED((_NPAD, _CH), F32),
        pltpu.VMEM_SHARED((_NPAD,), F32),
        pltpu.SemaphoreType.DMA,
        pltpu.SemaphoreType.DMA,
        pltpu.SemaphoreType.DMA,
        pltpu.SemaphoreType.DMA,
        pltpu.SemaphoreType.DMA,
      ],
  )
  def _edge_kernel(srcp, dstp, etp, as_hbm, ad_hbm, relm_hbm, h2_hbm,
                   acc_out, den_out, src_v, dst_v, et_v, ea_v, relm_v,
                   as_v, ad_v, rows_a, rows_b, zvec, h2_sh, acc_sh,
                   den_sh, sem_a, sem_b, sem_s0, sem_s1, sem_s2):
      cid = lax.axis_index("c")
      sid = lax.axis_index("s")

      zero16 = jnp.zeros((16,), F32)

      def _zrow_body(r, _):
          for c8 in range(_CH // 16):
              rows_a[r, pl.ds(c8 * 16, 16)] = zero16
          return 0
      lax.fori_loop(0, _ZROW, _zrow_body, 0)
      for k in range(640 // 16):
          zvec[pl.ds(k * 16, 16)] = zero16
      for b in range(5):
          off = sid * 640 + b * _ZROW
          pltpu.sync_copy(rows_a, acc_sh.at[pl.ds(off, _ZROW)])
      pltpu.sync_copy(zvec, den_sh.at[pl.ds(sid * 640, 640)])

      hoff = cid * _NPAD + sid * 640
      pltpu.sync_copy(h2_hbm.at[pl.ds(hoff, 640)],
                      h2_sh.at[pl.ds(sid * 640, 640)])
      pltpu.sync_copy(as_hbm, as_v)
      pltpu.sync_copy(ad_hbm, ad_v)
      pltpu.sync_copy(relm_hbm, relm_v)

      pltpu.sync_copy(srcp.at[sid, pl.ds(0, _NBC)], src_v.at[0])
      pltpu.sync_copy(dstp.at[sid, pl.ds(0, _NBC)], dst_v.at[0])
      pltpu.sync_copy(etp.at[sid, pl.ds(0, _NBC)], et_v.at[0])

      plsc.subcore_barrier()

      m = relm_v[...][15]

      def _p1(p, c, j):
          for k in range(_BE // 16):
              sl = pl.ds(k * 16, 16)
              sv = src_v[p, j, sl]
              dv = dst_v[p, j, sl]
              tv = et_v[p, j, sl]
              a = (plsc.load_gather(as_v, [sv])
                   + plsc.load_gather(ad_v, [dv])
                   + plsc.load_gather(relm_v, [tv]))
              a = jnp.where(a > 0.0, a, 0.2 * a)
              ea = jnp.exp(a - m)
              pos = (c * _NBC + j) * _BE + k * 16 + lax.iota(jnp.int32, 16)
              ea_v[p, j, sl] = jnp.where(pos < _EPT, ea, 0.0)

      def _proc(p, j, rows):
          def _scale(g, _):
              ev = ea_v[p, j, pl.ds(g * 16, 16)]
              for i in range(16):
                  cc = ev[i]
                  r = g * 16 + i
                  for c8 in range(_CH // 16):
                      sl = pl.ds(c8 * 16, 16)
                      rows[r, sl] = rows[r, sl] * cc
              return 0
          lax.fori_loop(0, _BE // 16, _scale, 0)
          pltpu.sync_copy(rows, acc_sh.at[dst_v.at[p, j]], add=True)
          pltpu.sync_copy(ea_v.at[p, j], den_sh.at[dst_v.at[p, j]], add=True)

      for c in range(_NCH):
          p = c % 2
          if c + 1 < _NCH:
              q = 1 - p
              nsl = pl.ds((c + 1) * _NBC, _NBC)
              pltpu.async_copy(srcp.at[sid, nsl], src_v.at[q], sem_s0)
              pltpu.async_copy(dstp.at[sid, nsl], dst_v.at[q], sem_s1)
              pltpu.async_copy(etp.at[sid, nsl], et_v.at[q], sem_s2)

          def _p1_body(j, _):
              _p1(p, c, j)
              return 0
          lax.fori_loop(0, _NBC, _p1_body, 0)

          pltpu.async_copy(h2_sh.at[src_v.at[p, 0]], rows_a, sem_a)

          def _pipe_body(jj, _):
              j0 = 2 * jj
              j1 = j0 + 1
              pltpu.async_copy(h2_sh.at[src_v.at[p, j1]], rows_b, sem_b)
              pltpu.make_async_copy(
                  h2_sh.at[pl.ds(0, _BE)], rows_a, sem_a).wait()
              _proc(p, j0, rows_a)

              @pl.when(jj + 1 < _NBC // 2)
              def _prefetch():
                  pltpu.async_copy(
                      h2_sh.at[src_v.at[p, j0 + 2]], rows_a, sem_a)

              pltpu.make_async_copy(
                  h2_sh.at[pl.ds(0, _BE)], rows_b, sem_b).wait()
              _proc(p, j1, rows_b)
              return 0
          lax.fori_loop(0, _NBC // 2, _pipe_body, 0)

          if c + 1 < _NCH:
              nsl = pl.ds((c + 1) * _NBC, _NBC)
              q = 1 - p
              pltpu.make_async_copy(
                  srcp.at[sid, nsl], src_v.at[q], sem_s0).wait()
              pltpu.make_async_copy(
                  dstp.at[sid, nsl], dst_v.at[q], sem_s1).wait()
              pltpu.make_async_copy(
                  etp.at[sid, nsl], et_v.at[q], sem_s2).wait()

      plsc.subcore_barrier()

      for b in range(5):
          off = sid * 640 + b * _ZROW
          pltpu.sync_copy(acc_sh.at[pl.ds(off, _ZROW)],
                          acc_out.at[pl.ds(cid * _NPAD + off, _ZROW)])
      pltpu.sync_copy(den_sh.at[pl.ds(sid * 640, 640)],
                      den_out.at[pl.ds(cid * _NPAD + sid * 640, 640)])

  return _edge_kernel


def _post_compute(has_res, accp_ref, denp_ref, bias_ref, hl_ref, nw_ref,
                  nb_ref):
    acc = jnp.concatenate(
        [accp_ref[0, 0:_N, :], accp_ref[1, 0:_N, :]], axis=1)
    den = denp_ref[0, 0:_N, :]
    o = acc / (den + 1e-16) + bias_ref[...]
    if has_res:
        o = o + hl_ref[...]
    o = o - jnp.mean(o)
    o = o / (jnp.sqrt(jnp.mean(o * o)) + 1e-5)
    o = o * nw_ref[...] + nb_ref[...]
    return jnp.maximum(o, 0.0)


def _postpre_body(has_res, *refs):
    if has_res:
        (accp_ref, denp_ref, bias_ref, hl_ref, nw_ref, nb_ref,
         w_ref, as_w_ref, ad_w_ref, ae_w_ref, we_ref, emb_ref,
         h2_ref, aso_ref, ado_ref, relm_ref, h_ref) = refs
    else:
        (accp_ref, denp_ref, bias_ref, nw_ref, nb_ref,
         w_ref, as_w_ref, ad_w_ref, ae_w_ref, we_ref, emb_ref,
         h2_ref, aso_ref, ado_ref, relm_ref, h_ref) = refs
        hl_ref = None
    hl = _post_compute(has_res, accp_ref, denp_ref, bias_ref, hl_ref,
                       nw_ref, nb_ref)
    h_ref[...] = hl
    h = jnp.dot(hl, w_ref[...], preferred_element_type=F32)
    h2_ref[0:_N, :] = h[:, 0:_CH]
    h2_ref[_N:_NPAD, :] = jnp.zeros((_NPAD - _N, _CH), F32)
    h2_ref[_NPAD:_NPAD + _N, :] = h[:, _CH:_C]
    h2_ref[_NPAD + _N:2 * _NPAD, :] = jnp.zeros((_NPAD - _N, _CH), F32)
    a_s = jnp.sum(h * as_w_ref[...], axis=1, keepdims=True)
    a_d = jnp.sum(h * ad_w_ref[...], axis=1, keepdims=True)
    aso_ref[0:_N, :] = a_s
    ado_ref[0:_N, :] = a_d
    aso_ref[_N:_NPAD, :] = jnp.zeros((_NPAD - _N, 1), F32)
    ado_ref[_N:_NPAD, :] = jnp.zeros((_NPAD - _N, 1), F32)
    wvec = jnp.sum(we_ref[...] * ae_w_ref[...], axis=1)
    rel = jnp.sum(emb_ref[...] * wvec[None, :], axis=1)
    m = jnp.maximum(jnp.max(a_s) + jnp.max(a_d) + jnp.max(rel), 0.0)
    vec = jnp.concatenate(
        [rel, jnp.zeros((16 - _NREL - 1,), F32), m[None]])
    relm_ref[...] = vec[None, :]


def _postpre_call(accp, denp, bias, hl, nw, nb, w, as_w, ad_w, ae_w, we,
                  emb):
    has_res = hl is not None
    args = ((accp, denp, bias) + ((hl,) if has_res else ()) + (nw, nb)
            + (w, as_w, ad_w, ae_w, we, emb))
    return pl.pallas_call(
        functools.partial(_postpre_body, has_res),
        out_shape=[
            jax.ShapeDtypeStruct((2 * _NPAD, _CH), F32),
            jax.ShapeDtypeStruct((_NPAD, 1), F32),
            jax.ShapeDtypeStruct((_NPAD, 1), F32),
            jax.ShapeDtypeStruct((1, 16), F32),
            jax.ShapeDtypeStruct((_N, _C), F32),
        ],
    )(*args)


def _postmlp_body(accp_ref, denp_ref, bias_ref, hl_ref, nw_ref, nb_ref,
                  w1_ref, b1_ref, w2_ref, b2_ref, o_ref):
    h = _post_compute(True, accp_ref, denp_ref, bias_ref, hl_ref,
                      nw_ref, nb_ref)
    z = jnp.dot(h, w1_ref[...], preferred_element_type=F32)
    z = jnp.maximum(z + b1_ref[...], 0.0)
    z = jnp.dot(z, w2_ref[...], preferred_element_type=F32) + b2_ref[...]
    o_ref[...] = 1.0 / (1.0 + jnp.exp(-z))


def _postmlp_call(accp, denp, bias, hl, nw, nb, w1, b1, w2, b2):
    return pl.pallas_call(
        _postmlp_body,
        out_shape=jax.ShapeDtypeStruct((_N, 1), F32),
    )(accp, denp, bias, hl, nw, nb, w1, b1, w2, b2)


def _pad_edges(a):
    return jnp.pad(a.reshape(_NS, _EPT),
                   ((0, 0), (0, _NB * _BE - _EPT))).reshape(_NS, _NB, _BE)


def kernel(x, edge_index, edge_type, lin_W, lin_edge_W, att_src, att_dst,
           att_edge, conv_bias, edge_emb, norm_weight, norm_bias,
           mlp_W1, mlp_b1, mlp_W2, mlp_b2):
    srcp = _pad_edges(edge_index[0])
    dstp = _pad_edges(edge_index[1])
    etp = _pad_edges(edge_type)
    nw = norm_weight[None, :]
    nb = norm_bias[None, :]

    edge_kernel = _edge_kernel_build()

    def run_edges(h2, aso, ado, relm):
        acc2, den2 = edge_kernel(
            srcp, dstp, etp, aso.reshape(_NPAD), ado.reshape(_NPAD),
            relm.reshape(16), h2)
        return acc2.reshape(2, _NPAD, _CH), den2.reshape(2, _NPAD, 1)

    h2, aso, ado, relm = _pre_call(
        True, x, lin_W[0], att_src[0][None, :], att_dst[0][None, :],
        att_edge[0][None, :], lin_edge_W[0], edge_emb[0])
    accp, denp = run_edges(h2, aso, ado, relm)

    hl = None
    for i in range(1, _L):
        h2, aso, ado, relm, hl = _postpre_call(
            accp, denp, conv_bias[i - 1][None, :], hl, nw, nb,
            lin_W[i], att_src[i][None, :], att_dst[i][None, :],
            att_edge[i][None, :], lin_edge_W[i], edge_emb[i])
        accp, denp = run_edges(h2, aso, ado, relm)

    return _postmlp_call(accp, denp, conv_bias[_L - 1][None, :], hl, nw, nb,
                         mlp_W1, mlp_b1[None, :], mlp_W2, mlp_b2[None, :])


# speedup vs baseline: 1.5854x; 1.0002x over previous
import functools

import jax
import jax.numpy as jnp
from jax import lax
from jax.experimental import pallas as pl
from jax.experimental.pallas import tpu as pltpu
from jax.experimental.pallas import tpu_sc as plsc

F32 = jnp.float32

_N = 10000
_C = 128
_E = 160000
_L = 3
_ED = 16
_NREL = 8

_NC = 2
_NS = 16
_CH = _C // _NC
_EPT = _E // _NS
_BE = 128
_NB = 80
_NBC = 8
_NCH = _NB // _NBC
_NPAD = 10240
_ZROW = 128


def _pre_body(first, hl_ref, w_ref, as_w_ref, ad_w_ref, ae_w_ref, we_ref,
              emb_ref, h2_ref, aso_ref, ado_ref, relm_ref):
    if first:
        colsum = jnp.sum(w_ref[...], axis=0, keepdims=True)
        h = hl_ref[...] * colsum
    else:
        h = jnp.dot(hl_ref[...], w_ref[...],
                    preferred_element_type=F32)
    h2_ref[0:_N, :] = h[:, 0:_CH]
    h2_ref[_N:_NPAD, :] = jnp.zeros((_NPAD - _N, _CH), F32)
    h2_ref[_NPAD:_NPAD + _N, :] = h[:, _CH:_C]
    h2_ref[_NPAD + _N:2 * _NPAD, :] = jnp.zeros((_NPAD - _N, _CH), F32)
    a_s = jnp.sum(h * as_w_ref[...], axis=1, keepdims=True)
    a_d = jnp.sum(h * ad_w_ref[...], axis=1, keepdims=True)
    aso_ref[0:_N, :] = a_s
    ado_ref[0:_N, :] = a_d
    aso_ref[_N:_NPAD, :] = jnp.zeros((_NPAD - _N, 1), F32)
    ado_ref[_N:_NPAD, :] = jnp.zeros((_NPAD - _N, 1), F32)
    wvec = jnp.sum(we_ref[...] * ae_w_ref[...], axis=1)
    rel = jnp.sum(emb_ref[...] * wvec[None, :], axis=1)
    m = jnp.maximum(jnp.max(a_s) + jnp.max(a_d) + jnp.max(rel), 0.0)
    vec = jnp.concatenate(
        [rel, jnp.zeros((16 - _NREL - 1,), F32), m[None]])
    relm_ref[...] = vec[None, :]


def _pre_call(first, hl, w, as_w, ad_w, ae_w, we, emb):
    return pl.pallas_call(
        functools.partial(_pre_body, first),
        out_shape=[
            jax.ShapeDtypeStruct((2 * _NPAD, _CH), F32),
            jax.ShapeDtypeStruct((_NPAD, 1), F32),
            jax.ShapeDtypeStruct((_NPAD, 1), F32),
            jax.ShapeDtypeStruct((1, 16), F32),
        ],
    )(hl, w, as_w, ad_w, ae_w, we, emb)


@functools.cache
def _edge_kernel_build():
  mesh = plsc.VectorSubcoreMesh(core_axis_name="c", subcore_axis_name="s",
                                num_cores=_NC, num_subcores=_NS)

  @functools.partial(
    pl.kernel,
    out_type=[
        jax.ShapeDtypeStruct((2 * _NPAD, _CH), F32),
        jax.ShapeDtypeStruct((2 * _NPAD,), F32),
    ],
    mesh=mesh,
    compiler_params=pltpu.CompilerParams(needs_layout_passes=False,
                                         use_tc_tiling_on_sc=False),
    scratch_types=[
        pltpu.VMEM((2, _NBC, _BE), jnp.int32),
        pltpu.VMEM((2, _NBC, _BE), jnp.int32),
        pltpu.VMEM((2, _NBC, _BE), jnp.int32),
        pltpu.VMEM((2, _NBC, _BE), F32),
        pltpu.VMEM((16,), F32),
        pltpu.VMEM((_NPAD,), F32),
        pltpu.VMEM((_NPAD,), F32),
        pltpu.VMEM((_BE, _CH), F32),
        pltpu.VMEM((_BE, _CH), F32),
        pltpu.VMEM((640,), F32),
        pltpu.VMEM_SHARED((_NPAD, _CH), F32),
        pltpu.VMEM_SHARED((_NPAD, _CH), F32),
        pltpu.VMEM_SHARED((_NPAD,), F32),
        pltpu.SemaphoreType.DMA,
        pltpu.SemaphoreType.DMA,
        pltpu.SemaphoreType.DMA,
        pltpu.SemaphoreType.DMA,
        pltpu.SemaphoreType.DMA,
      ],
  )
  def _edge_kernel(srcp, dstp, etp, as_hbm, ad_hbm, relm_hbm, h2_hbm,
                   acc_out, den_out, src_v, dst_v, et_v, ea_v, relm_v,
                   as_v, ad_v, rows_a, rows_b, zvec, h2_sh, acc_sh,
                   den_sh, sem_a, sem_b, sem_s0, sem_s1, sem_s2):
      cid = lax.axis_index("c")
      sid = lax.axis_index("s")

      zero16 = jnp.zeros((16,), F32)

      def _zrow_body(r, _):
          for c8 in range(_CH // 16):
              rows_a[r, pl.ds(c8 * 16, 16)] = zero16
          return 0
      lax.fori_loop(0, _ZROW, _zrow_body, 0)
      for k in range(640 // 16):
          zvec[pl.ds(k * 16, 16)] = zero16
      for b in range(5):
          off = sid * 640 + b * _ZROW
          pltpu.sync_copy(rows_a, acc_sh.at[pl.ds(off, _ZROW)])
      pltpu.sync_copy(zvec, den_sh.at[pl.ds(sid * 640, 640)])

      hoff = cid * _NPAD + sid * 640
      pltpu.sync_copy(h2_hbm.at[pl.ds(hoff, 640)],
                      h2_sh.at[pl.ds(sid * 640, 640)])
      pltpu.sync_copy(as_hbm, as_v)
      pltpu.sync_copy(ad_hbm, ad_v)
      pltpu.sync_copy(relm_hbm, relm_v)

      pltpu.sync_copy(srcp.at[sid, pl.ds(0, _NBC)], src_v.at[0])
      pltpu.sync_copy(dstp.at[sid, pl.ds(0, _NBC)], dst_v.at[0])
      pltpu.sync_copy(etp.at[sid, pl.ds(0, _NBC)], et_v.at[0])

      plsc.subcore_barrier()

      m = relm_v[...][15]

      def _p1(p, c, j):
          for k in range(_BE // 16):
              sl = pl.ds(k * 16, 16)
              sv = src_v[p, j, sl]
              dv = dst_v[p, j, sl]
              tv = et_v[p, j, sl]
              a = (plsc.load_gather(as_v, [sv])
                   + plsc.load_gather(ad_v, [dv])
                   + plsc.load_gather(relm_v, [tv]))
              a = jnp.where(a > 0.0, a, 0.2 * a)
              ea = jnp.exp(a - m)
              pos = (c * _NBC + j) * _BE + k * 16 + lax.iota(jnp.int32, 16)
              ea_v[p, j, sl] = jnp.where(pos < _EPT, ea, 0.0)

      def _proc(p, j, rows):
          def _scale(g, _):
              ev = ea_v[p, j, pl.ds(g * 16, 16)]
              for i in range(16):
                  cc = ev[i]
                  r = g * 16 + i
                  for c8 in range(_CH // 16):
                      sl = pl.ds(c8 * 16, 16)
                      rows[r, sl] = rows[r, sl] * cc
              return 0
          lax.fori_loop(0, _BE // 16, _scale, 0)
          pltpu.sync_copy(rows, acc_sh.at[dst_v.at[p, j]], add=True)
          pltpu.sync_copy(ea_v.at[p, j], den_sh.at[dst_v.at[p, j]], add=True)

      for c in range(_NCH):
          p = c % 2
          if c + 1 < _NCH:
              q = 1 - p
              nsl = pl.ds((c + 1) * _NBC, _NBC)
              pltpu.async_copy(srcp.at[sid, nsl], src_v.at[q], sem_s0)
              pltpu.async_copy(dstp.at[sid, nsl], dst_v.at[q], sem_s1)
              pltpu.async_copy(etp.at[sid, nsl], et_v.at[q], sem_s2)

          def _p1_body(j, _):
              _p1(p, c, j)
              return 0
          lax.fori_loop(0, _NBC, _p1_body, 0)

          pltpu.async_copy(h2_sh.at[src_v.at[p, 0]], rows_a, sem_a)

          def _pipe_body(jj, _):
              j0 = 2 * jj
              j1 = j0 + 1
              pltpu.async_copy(h2_sh.at[src_v.at[p, j1]], rows_b, sem_b)
              pltpu.make_async_copy(
                  h2_sh.at[pl.ds(0, _BE)], rows_a, sem_a).wait()
              _proc(p, j0, rows_a)

              @pl.when(jj + 1 < _NBC // 2)
              def _prefetch():
                  pltpu.async_copy(
                      h2_sh.at[src_v.at[p, j0 + 2]], rows_a, sem_a)

              pltpu.make_async_copy(
                  h2_sh.at[pl.ds(0, _BE)], rows_b, sem_b).wait()
              _proc(p, j1, rows_b)
              return 0
          lax.fori_loop(0, _NBC // 2, _pipe_body, 0)

          if c + 1 < _NCH:
              nsl = pl.ds((c + 1) * _NBC, _NBC)
              q = 1 - p
              pltpu.make_async_copy(
                  srcp.at[sid, nsl], src_v.at[q], sem_s0).wait()
              pltpu.make_async_copy(
                  dstp.at[sid, nsl], dst_v.at[q], sem_s1).wait()
              pltpu.make_async_copy(
                  etp.at[sid, nsl], et_v.at[q], sem_s2).wait()

      plsc.subcore_barrier()

      for b in range(5):
          off = sid * 640 + b * _ZROW
          pltpu.sync_copy(acc_sh.at[pl.ds(off, _ZROW)],
                          acc_out.at[pl.ds(cid * _NPAD + off, _ZROW)])
      pltpu.sync_copy(den_sh.at[pl.ds(sid * 640, 640)],
                      den_out.at[pl.ds(cid * _NPAD + sid * 640, 640)])

  return _edge_kernel


def _post_compute(has_res, accp_ref, denp_ref, bias_ref, hl_ref, nw_ref,
                  nb_ref):
    acc = jnp.concatenate(
        [accp_ref[0, 0:_N, :], accp_ref[1, 0:_N, :]], axis=1)
    den = denp_ref[0, 0:_N, :]
    o = acc / (den + 1e-16) + bias_ref[...]
    if has_res:
        o = o + hl_ref[...]
    o = o - jnp.mean(o)
    o = o / (jnp.sqrt(jnp.mean(o * o)) + 1e-5)
    o = o * nw_ref[...] + nb_ref[...]
    return jnp.maximum(o, 0.0)


def _postpre_body(has_res, *refs):
    if has_res:
        (accp_ref, denp_ref, bias_ref, hl_ref, nw_ref, nb_ref,
         w_ref, as_w_ref, ad_w_ref, ae_w_ref, we_ref, emb_ref,
         h2_ref, aso_ref, ado_ref, relm_ref, h_ref) = refs
    else:
        (accp_ref, denp_ref, bias_ref, nw_ref, nb_ref,
         w_ref, as_w_ref, ad_w_ref, ae_w_ref, we_ref, emb_ref,
         h2_ref, aso_ref, ado_ref, relm_ref, h_ref) = refs
        hl_ref = None
    hl = _post_compute(has_res, accp_ref, denp_ref, bias_ref, hl_ref,
                       nw_ref, nb_ref)
    h_ref[...] = hl
    h = jnp.dot(hl, w_ref[...], preferred_element_type=F32)
    h2_ref[0:_N, :] = h[:, 0:_CH]
    h2_ref[_N:_NPAD, :] = jnp.zeros((_NPAD - _N, _CH), F32)
    h2_ref[_NPAD:_NPAD + _N, :] = h[:, _CH:_C]
    h2_ref[_NPAD + _N:2 * _NPAD, :] = jnp.zeros((_NPAD - _N, _CH), F32)
    a_s = jnp.sum(h * as_w_ref[...], axis=1, keepdims=True)
    a_d = jnp.sum(h * ad_w_ref[...], axis=1, keepdims=True)
    aso_ref[0:_N, :] = a_s
    ado_ref[0:_N, :] = a_d
    aso_ref[_N:_NPAD, :] = jnp.zeros((_NPAD - _N, 1), F32)
    ado_ref[_N:_NPAD, :] = jnp.zeros((_NPAD - _N, 1), F32)
    wvec = jnp.sum(we_ref[...] * ae_w_ref[...], axis=1)
    rel = jnp.sum(emb_ref[...] * wvec[None, :], axis=1)
    m = jnp.maximum(jnp.max(a_s) + jnp.max(a_d) + jnp.max(rel), 0.0)
    vec = jnp.concatenate(
        [rel, jnp.zeros((16 - _NREL - 1,), F32), m[None]])
    relm_ref[...] = vec[None, :]


def _postpre_call(accp, denp, bias, hl, nw, nb, w, as_w, ad_w, ae_w, we,
                  emb):
    has_res = hl is not None
    args = ((accp, denp, bias) + ((hl,) if has_res else ()) + (nw, nb)
            + (w, as_w, ad_w, ae_w, we, emb))
    return pl.pallas_call(
        functools.partial(_postpre_body, has_res),
        out_shape=[
            jax.ShapeDtypeStruct((2 * _NPAD, _CH), F32),
            jax.ShapeDtypeStruct((_NPAD, 1), F32),
            jax.ShapeDtypeStruct((_NPAD, 1), F32),
            jax.ShapeDtypeStruct((1, 16), F32),
            jax.ShapeDtypeStruct((_N, _C), F32),
        ],
    )(*args)


def _postmlp_body(accp_ref, denp_ref, bias_ref, hl_ref, nw_ref, nb_ref,
                  w1_ref, b1_ref, w2_ref, b2_ref, o_ref):
    h = _post_compute(True, accp_ref, denp_ref, bias_ref, hl_ref,
                      nw_ref, nb_ref)
    z = jnp.dot(h, w1_ref[...], preferred_element_type=F32)
    z = jnp.maximum(z + b1_ref[...], 0.0)
    z = jnp.dot(z, w2_ref[...], preferred_element_type=F32) + b2_ref[...]
    o_ref[...] = 1.0 / (1.0 + jnp.exp(-z))


def _postmlp_call(accp, denp, bias, hl, nw, nb, w1, b1, w2, b2):
    return pl.pallas_call(
        _postmlp_body,
        out_shape=jax.ShapeDtypeStruct((_N, 1), F32),
    )(accp, denp, bias, hl, nw, nb, w1, b1, w2, b2)


def _pad_edges(a):
    return jnp.pad(a.reshape(_NS, _EPT),
                   ((0, 0), (0, _NB * _BE - _EPT))).reshape(_NS, _NB, _BE)


def kernel(x, edge_index, edge_type, lin_W, lin_edge_W, att_src, att_dst,
           att_edge, conv_bias, edge_emb, norm_weight, norm_bias,
           mlp_W1, mlp_b1, mlp_W2, mlp_b2):
    srcp = _pad_edges(edge_index[0])
    dstp = _pad_edges(edge_index[1])
    etp = _pad_edges(edge_type)
    nw = norm_weight[None, :]
    nb = norm_bias[None, :]

    edge_kernel = _edge_kernel_build()

    def run_edges(h2, aso, ado, relm):
        acc2, den2 = edge_kernel(
            srcp, dstp, etp, aso.reshape(_NPAD), ado.reshape(_NPAD),
            relm.reshape(16), h2.reshape(2 * _NPAD, _CH))
        return acc2.reshape(2, _NPAD, _CH), den2.reshape(2, _NPAD, 1)

    h2, aso, ado, relm = _pre_call(
        True, x, lin_W[0], att_src[0][None, :], att_dst[0][None, :],
        att_edge[0][None, :], lin_edge_W[0], edge_emb[0])
    accp, denp = run_edges(h2, aso, ado, relm)

    hl = None
    for i in range(1, _L):
        h2, aso, ado, relm, hl = _postpre_call(
            accp, denp, conv_bias[i - 1][None, :], hl, nw, nb,
            lin_W[i], att_src[i][None, :], att_dst[i][None, :],
            att_edge[i][None, :], lin_edge_W[i], edge_emb[i])
        accp, denp = run_edges(h2, aso, ado, relm)

    return _postmlp_call(accp, denp, conv_bias[_L - 1][None, :], hl, nw, nb,
                         mlp_W1, mlp_b1[None, :], mlp_W2, mlp_b2[None, :])
